# Initial kernel scaffold; baseline (speedup 1.0000x reference)
import jax
import jax.numpy as jnp
from jax.experimental import pallas as pl


def kernel(x, edge_index, pseudo, params, Wout, bout):
    raise NotImplementedError("write your pallas kernel here")


# speedup vs baseline: 1.7482x; 1.7482x over previous
import functools

import jax
import jax.numpy as jnp
from jax import lax
from jax.experimental import pallas as pl
from jax.experimental.pallas import tpu as pltpu
from jax.experimental.pallas import tpu_sc as plsc

N = 10000
E = 160000
K = 4
EPS = 1e-14

NC = 2
NS = 16
NW = NC * NS

EP = 163840
BATCHES = EP // (NW * 128)
PER_W = BATCHES * 128
ACC_R = 10112
ZROWS = ACC_R // NS
EB = 2560

_MESH = plsc.VectorSubcoreMesh(core_axis_name="c", subcore_axis_name="s")


@functools.partial(
    pl.kernel,
    out_type=jax.ShapeDtypeStruct((EP, 128), jnp.float32),
    mesh=_MESH,
    scratch_types=[
        pltpu.VMEM((BATCHES, 128), jnp.int32),
        pltpu.VMEM((128, 128), jnp.float32),
        pltpu.SemaphoreType.DMA,
    ],
)
def _sc_gather(h_hbm, idx_hbm, out_hbm, idx_v, rows_v, sem):
    cid = lax.axis_index("c")
    sid = lax.axis_index("s")
    w = sid * NC + cid
    pltpu.sync_copy(idx_hbm.at[pl.ds(w * BATCHES, BATCHES)], idx_v)

    def body(j, carry):
        pltpu.async_copy(h_hbm.at[idx_v.at[j]], rows_v, sem).wait()
        pltpu.sync_copy(rows_v, out_hbm.at[pl.ds(w * PER_W + j * 128, 128)])
        return carry

    lax.fori_loop(0, BATCHES, body, 0)


def _make_sc_scatter(n_ch):
    @functools.partial(
        pl.kernel,
        out_type=jax.ShapeDtypeStruct((n_ch * 2 * ACC_R, 128), jnp.float32),
        mesh=_MESH,
        scratch_types=[
            pltpu.VMEM((BATCHES, 128), jnp.int32),
            pltpu.VMEM((128, 128), jnp.float32),
            pltpu.VMEM_SHARED((ACC_R, 128), jnp.float32),
        ],
    )
    def _sc_scatter(msg_hbm, idx_hbm, zeros_hbm, out_hbm, idx_v, msg_v, acc):
        cid = lax.axis_index("c")
        sid = lax.axis_index("s")
        w = sid * NC + cid
        pltpu.sync_copy(idx_hbm.at[pl.ds(w * BATCHES, BATCHES)], idx_v)
        for c in range(n_ch):
            pltpu.sync_copy(zeros_hbm, acc.at[pl.ds(sid * ZROWS, ZROWS)])
            plsc.subcore_barrier()

            def body(j, carry):
                base = c * EP + w * PER_W + j * 128
                pltpu.sync_copy(msg_hbm.at[pl.ds(base, 128)], msg_v)
                pltpu.sync_copy(msg_v, acc.at[idx_v.at[j]], add=True)
                return carry

            lax.fori_loop(0, BATCHES, body, 0)
            plsc.subcore_barrier()
            out_base = (c * 2 + cid) * ACC_R + sid * ZROWS
            pltpu.sync_copy(acc.at[pl.ds(sid * ZROWS, ZROWS)],
                            out_hbm.at[pl.ds(out_base, ZROWS)])
            plsc.subcore_barrier()

    return _sc_scatter


_sc_scatter_1 = _make_sc_scatter(1)
_sc_scatter_2 = _make_sc_scatter(2)


def _gauss_body(ps_ref, mu0_ref, mu1_ref, s0_ref, s1_ref, out_ref):
    p0 = ps_ref[:, 0:1]
    p1 = ps_ref[:, 1:2]
    d0 = p0 - mu0_ref[...]
    d1 = p1 - mu1_ref[...]
    i0 = 1.0 / (EPS + s0_ref[...] * s0_ref[...])
    i1 = 1.0 / (EPS + s1_ref[...] * s1_ref[...])
    out_ref[...] = jnp.exp(-0.5 * (d0 * d0 * i0 + d1 * d1 * i1))


def _gauss_all(pseudo_p, mu0, mu1, s0, s1, nl):
    return pl.pallas_call(
        _gauss_body,
        grid=(EP // EB,),
        in_specs=[
            pl.BlockSpec((EB, 2), lambda i: (i, 0)),
            pl.BlockSpec((1, K * nl), lambda i: (0, 0)),
            pl.BlockSpec((1, K * nl), lambda i: (0, 0)),
            pl.BlockSpec((1, K * nl), lambda i: (0, 0)),
            pl.BlockSpec((1, K * nl), lambda i: (0, 0)),
        ],
        out_specs=pl.BlockSpec((EB, K * nl), lambda i: (i, 0)),
        out_shape=jax.ShapeDtypeStruct((EP, K * nl), jnp.float32),
    )(pseudo_p, mu0, mu1, s0, s1)


def _make_msg(layer, n_ch, nl):
    fout_p = n_ch * 128

    def body(xj_ref, g_ref, wg_ref, out_ref):
        xj = xj_ref[...]
        acc = jnp.zeros((EB, fout_p), jnp.float32)
        for k in range(K):
            gk = g_ref[:, K * layer + k:K * layer + k + 1]
            acc = acc + jnp.dot(xj, wg_ref[k], preferred_element_type=jnp.float32) * gk
        if layer == 0:
            lane = lax.broadcasted_iota(jnp.int32, (1, fout_p), 1)
            acc = acc + jnp.where(lane == 16, 1.0, 0.0)
        for c in range(n_ch):
            out_ref[c] = acc[:, c * 128:(c + 1) * 128]

    def run(xj, gauss, wg):
        out = pl.pallas_call(
            body,
            grid=(EP // EB,),
            in_specs=[
                pl.BlockSpec((EB, 128), lambda i: (i, 0)),
                pl.BlockSpec((EB, K * nl), lambda i: (i, 0)),
                pl.BlockSpec((K, 128, fout_p), lambda i: (0, 0, 0)),
            ],
            out_specs=pl.BlockSpec((n_ch, EB, 128), lambda i: (0, i, 0)),
            out_shape=jax.ShapeDtypeStruct((n_ch, EP, 128), jnp.float32),
        )(xj, gauss, wg)
        return out.reshape(n_ch * EP, 128)

    return run


def _make_finalize(layer, n_ch, fout):
    first = layer == 0
    out_w = 128 if fout < 128 else fout

    def body(*refs):
        if first:
            scat_ref, h_ref, wroot_ref, bias_ref, gamma_ref, beta_ref, out_ref, deg_ref = refs
        else:
            scat_ref, deg_in_ref, h_ref, wroot_ref, bias_ref, gamma_ref, beta_ref, out_ref = refs
        parts = []
        for c in range(n_ch):
            a = (scat_ref[(c * 2) * ACC_R:(c * 2) * ACC_R + N, :]
                 + scat_ref[(c * 2 + 1) * ACC_R:(c * 2 + 1) * ACC_R + N, :])
            parts.append(a)
        agg = parts[0] if n_ch == 1 else jnp.concatenate(parts, axis=1)
        if first:
            deg = agg[:, 16:17]
            deg_ref[...] = deg
        else:
            deg = deg_in_ref[...]
        agg = agg[:, :fout] / jnp.clip(deg, 1.0, None)
        r = agg + jnp.dot(h_ref[...], wroot_ref[...],
                          preferred_element_type=jnp.float32) + bias_ref[...]
        m = jnp.mean(r, axis=0, keepdims=True)
        v = jnp.mean((r - m) * (r - m), axis=0, keepdims=True)
        hn = gamma_ref[...] * (r - m) * lax.rsqrt(v + 1e-5) + beta_ref[...]
        hn = jnp.maximum(hn, 0.0)
        if out_w > fout:
            hn = jnp.concatenate(
                [hn, jnp.zeros((N, out_w - fout), jnp.float32)], axis=1)
        out_ref[...] = hn

    def run(scat, deg, h, wroot, bias, gamma, beta):
        full = lambda s: pl.BlockSpec(s, lambda: tuple(0 for _ in s))
        in_specs = [full((n_ch * 2 * ACC_R, 128))]
        args = [scat]
        if not first:
            in_specs.append(full((N, 1)))
            args.append(deg)
        in_specs += [full((N, 128)), full((128, fout)),
                     full((1, fout)), full((1, fout)), full((1, fout))]
        args += [h, wroot, bias, gamma, beta]
        out_shape = [jax.ShapeDtypeStruct((N, out_w), jnp.float32)]
        out_specs = [full((N, out_w))]
        if first:
            out_shape.append(jax.ShapeDtypeStruct((N, 1), jnp.float32))
            out_specs.append(full((N, 1)))
        res = pl.pallas_call(
            body,
            in_specs=in_specs,
            out_specs=out_specs,
            out_shape=out_shape,
        )(*args)
        return (res[0], res[1]) if first else (res[0], deg)

    return run


def _head_body(h_ref, wout_ref, bout_ref, out_ref):
    pooled = jnp.mean(h_ref[...], axis=0, keepdims=True)
    logits = jnp.dot(pooled, wout_ref[...],
                     preferred_element_type=jnp.float32) + bout_ref[...]
    mx = jnp.max(logits, axis=1, keepdims=True)
    lse = jnp.log(jnp.sum(jnp.exp(logits - mx), axis=1, keepdims=True)) + mx
    out_ref[...] = (logits - lse)[:, :10]


def _head(h, wout_p, bout_p, fin):
    full = lambda s: pl.BlockSpec(s, lambda: tuple(0 for _ in s))
    return pl.pallas_call(
        _head_body,
        in_specs=[full((N, fin)), full((fin, 128)), full((1, 128))],
        out_specs=full((1, 10)),
        out_shape=jax.ShapeDtypeStruct((1, 10), jnp.float32),
    )(h, wout_p, bout_p)


def kernel(x, edge_index, pseudo, params, Wout, bout):
    nl = len(params)
    src = edge_index[0]
    dst = edge_index[1]

    pad = EP - E
    src_p = jnp.concatenate([src, jnp.zeros((pad,), jnp.int32)])
    trash = N + (jnp.arange(pad, dtype=jnp.int32) % 16)
    dst_p = jnp.concatenate([dst, trash])
    src2d = src_p.reshape(EP // 128, 128)
    dst2d = dst_p.reshape(EP // 128, 128)
    pseudo_p = jnp.concatenate([pseudo, jnp.zeros((pad, 2), jnp.float32)])

    mu = jnp.stack([p["mu"] for p in params])
    sg = jnp.stack([p["sigma"] for p in params])
    mu0 = mu[:, :, 0].reshape(1, nl * K)
    mu1 = mu[:, :, 1].reshape(1, nl * K)
    s0 = sg[:, :, 0].reshape(1, nl * K)
    s1 = sg[:, :, 1].reshape(1, nl * K)
    gauss = _gauss_all(pseudo_p, mu0, mu1, s0, s1, nl)

    zeros = jnp.zeros((ZROWS, 128), jnp.float32)

    h = jnp.pad(x, ((0, 0), (0, 128 - x.shape[1])))
    deg = None
    for l, p in enumerate(params):
        fin = p["Wroot"].shape[0]
        fout = p["Wroot"].shape[1]
        n_ch = 2 if fout > 128 else 1
        wg = p["Wg"].reshape(fin, K, fout).transpose(1, 0, 2)
        wg = jnp.pad(wg, ((0, 0), (0, 128 - fin), (0, n_ch * 128 - fout)))
        wroot = jnp.pad(p["Wroot"], ((0, 128 - fin), (0, 0)))

        xj = _sc_gather(h, src2d)
        msg = _make_msg(l, n_ch, nl)(xj, gauss, wg)
        scat = (_sc_scatter_1 if n_ch == 1 else _sc_scatter_2)(msg, dst2d, zeros)
        h, deg = _make_finalize(l, n_ch, fout)(
            scat, deg, h, wroot,
            p["bias"].reshape(1, fout),
            p["gamma"].reshape(1, fout),
            p["beta"].reshape(1, fout),
        )

    wout_p = jnp.pad(Wout, ((0, 0), (0, 128 - Wout.shape[1])))
    bout_p = jnp.full((1, 128), -1e30, jnp.float32)
    bout_p = bout_p.at[0, :10].set(bout)
    return _head(h, wout_p, bout_p, h.shape[1])


# speedup vs baseline: 2.0228x; 1.1571x over previous
import functools

import jax
import jax.numpy as jnp
from jax import lax
from jax.experimental import pallas as pl
from jax.experimental.pallas import tpu as pltpu
from jax.experimental.pallas import tpu_sc as plsc

N = 10000
E = 160000
K = 4
EPS = 1e-14

NC = 2
NS = 16
NW = NC * NS

EP = 163840
BATCHES = EP // (NW * 128)
PER_W = BATCHES * 128
ACC_R = 10112
ZROWS = ACC_R // NS
EB = 2560

_MESH = plsc.VectorSubcoreMesh(core_axis_name="c", subcore_axis_name="s")


NBUF = 4
NBUF_S = 2


@functools.partial(
    pl.kernel,
    out_type=jax.ShapeDtypeStruct((EP, 128), jnp.float32),
    mesh=_MESH,
    scratch_types=[
        pltpu.VMEM((BATCHES, 128), jnp.int32),
    ] + [pltpu.VMEM((128, 128), jnp.float32) for _ in range(NBUF)]
      + [pltpu.SemaphoreType.DMA for _ in range(2 * NBUF)],
)
def _sc_gather(h_hbm, idx_hbm, out_hbm, idx_v, *bufsem):
    bufs = bufsem[:NBUF]
    gsem = bufsem[NBUF:2 * NBUF]
    osem = bufsem[2 * NBUF:]
    cid = lax.axis_index("c")
    sid = lax.axis_index("s")
    w = sid * NC + cid
    pltpu.sync_copy(idx_hbm.at[pl.ds(w * BATCHES, BATCHES)], idx_v)

    gh = [None] * NBUF
    oh = [None] * NBUF
    for j in range(NBUF):
        gh[j % NBUF] = pltpu.async_copy(
            h_hbm.at[idx_v.at[j]], bufs[j % NBUF], gsem[j % NBUF])
    for j in range(BATCHES):
        b = j % NBUF
        gh[b].wait()
        oh[b] = pltpu.async_copy(
            bufs[b], out_hbm.at[pl.ds(w * PER_W + j * 128, 128)], osem[b])
        if j + NBUF < BATCHES:
            oh[b].wait()
            gh[b] = pltpu.async_copy(
                h_hbm.at[idx_v.at[j + NBUF]], bufs[b], gsem[b])
    for j in range(BATCHES - NBUF, BATCHES):
        oh[j % NBUF].wait()


def _make_sc_scatter(n_ch):
    @functools.partial(
        pl.kernel,
        out_type=jax.ShapeDtypeStruct((n_ch * 2 * ACC_R, 128), jnp.float32),
        mesh=_MESH,
        scratch_types=[
            pltpu.VMEM((BATCHES, 128), jnp.int32),
            pltpu.VMEM_SHARED((ACC_R, 128), jnp.float32),
        ] + [pltpu.VMEM((128, 128), jnp.float32) for _ in range(NBUF_S)]
          + [pltpu.SemaphoreType.DMA for _ in range(2 * NBUF_S)],
    )
    def _sc_scatter(msg_hbm, idx_hbm, zeros_hbm, out_hbm, idx_v, acc, *bufsem):
        bufs = bufsem[:NBUF_S]
        isem = bufsem[NBUF_S:2 * NBUF_S]
        asem = bufsem[2 * NBUF_S:]
        cid = lax.axis_index("c")
        sid = lax.axis_index("s")
        w = sid * NC + cid
        pltpu.sync_copy(idx_hbm.at[pl.ds(w * BATCHES, BATCHES)], idx_v)
        for c in range(n_ch):
            pltpu.sync_copy(zeros_hbm.at[pl.ds(sid * ZROWS, ZROWS)],
                            acc.at[pl.ds(sid * ZROWS, ZROWS)])
            plsc.subcore_barrier()
            ih = [None] * NBUF_S
            ah = [None] * NBUF_S
            for j in range(NBUF_S):
                base = c * EP + w * PER_W + j * 128
                ih[j] = pltpu.async_copy(
                    msg_hbm.at[pl.ds(base, 128)], bufs[j], isem[j])
            for j in range(BATCHES):
                b = j % NBUF_S
                ih[b].wait()
                ah[b] = pltpu.async_copy(
                    bufs[b], acc.at[idx_v.at[j]], asem[b], add=True)
                if j + NBUF_S < BATCHES:
                    ah[b].wait()
                    base = c * EP + w * PER_W + (j + NBUF_S) * 128
                    ih[b] = pltpu.async_copy(
                        msg_hbm.at[pl.ds(base, 128)], bufs[b], isem[b])
            for j in range(BATCHES - NBUF_S, BATCHES):
                ah[j % NBUF_S].wait()
            plsc.subcore_barrier()
            out_base = (c * 2 + cid) * ACC_R + sid * ZROWS
            pltpu.sync_copy(acc.at[pl.ds(sid * ZROWS, ZROWS)],
                            out_hbm.at[pl.ds(out_base, ZROWS)])
            plsc.subcore_barrier()

    return _sc_scatter


_sc_scatter_1 = _make_sc_scatter(1)
_sc_scatter_2 = _make_sc_scatter(2)


def _gauss_body(ps_ref, mu0_ref, mu1_ref, s0_ref, s1_ref, out_ref):
    p0 = ps_ref[:, 0:1]
    p1 = ps_ref[:, 1:2]
    d0 = p0 - mu0_ref[...]
    d1 = p1 - mu1_ref[...]
    i0 = 1.0 / (EPS + s0_ref[...] * s0_ref[...])
    i1 = 1.0 / (EPS + s1_ref[...] * s1_ref[...])
    out_ref[...] = jnp.exp(-0.5 * (d0 * d0 * i0 + d1 * d1 * i1))


def _gauss_all(pseudo_p, mu0, mu1, s0, s1, nl):
    return pl.pallas_call(
        _gauss_body,
        grid=(EP // EB,),
        in_specs=[
            pl.BlockSpec((EB, 2), lambda i: (i, 0)),
            pl.BlockSpec((1, K * nl), lambda i: (0, 0)),
            pl.BlockSpec((1, K * nl), lambda i: (0, 0)),
            pl.BlockSpec((1, K * nl), lambda i: (0, 0)),
            pl.BlockSpec((1, K * nl), lambda i: (0, 0)),
        ],
        out_specs=pl.BlockSpec((EB, K * nl), lambda i: (i, 0)),
        out_shape=jax.ShapeDtypeStruct((EP, K * nl), jnp.float32),
    )(pseudo_p, mu0, mu1, s0, s1)


def _make_msg(layer, n_ch, nl):
    fout_p = n_ch * 128

    def body(xj_ref, g_ref, wg_ref, out_ref):
        xj = xj_ref[...]
        acc = jnp.zeros((EB, fout_p), jnp.float32)
        for k in range(K):
            gk = g_ref[:, K * layer + k:K * layer + k + 1]
            acc = acc + jnp.dot(xj, wg_ref[k], preferred_element_type=jnp.float32) * gk
        if layer == 0:
            lane = lax.broadcasted_iota(jnp.int32, (1, fout_p), 1)
            acc = acc + jnp.where(lane == 16, 1.0, 0.0)
        for c in range(n_ch):
            out_ref[c] = acc[:, c * 128:(c + 1) * 128]

    def run(xj, gauss, wg):
        out = pl.pallas_call(
            body,
            grid=(EP // EB,),
            in_specs=[
                pl.BlockSpec((EB, 128), lambda i: (i, 0)),
                pl.BlockSpec((EB, K * nl), lambda i: (i, 0)),
                pl.BlockSpec((K, 128, fout_p), lambda i: (0, 0, 0)),
            ],
            out_specs=pl.BlockSpec((n_ch, EB, 128), lambda i: (0, i, 0)),
            out_shape=jax.ShapeDtypeStruct((n_ch, EP, 128), jnp.float32),
        )(xj, gauss, wg)
        return out.reshape(n_ch * EP, 128)

    return run


def _make_finalize(layer, n_ch, fout):
    first = layer == 0
    out_w = 128 if fout < 128 else fout

    def body(*refs):
        if first:
            scat_ref, h_ref, wroot_ref, bias_ref, gamma_ref, beta_ref, out_ref, deg_ref = refs
        else:
            scat_ref, deg_in_ref, h_ref, wroot_ref, bias_ref, gamma_ref, beta_ref, out_ref = refs
        parts = []
        for c in range(n_ch):
            a = (scat_ref[(c * 2) * ACC_R:(c * 2) * ACC_R + N, :]
                 + scat_ref[(c * 2 + 1) * ACC_R:(c * 2 + 1) * ACC_R + N, :])
            parts.append(a)
        agg = parts[0] if n_ch == 1 else jnp.concatenate(parts, axis=1)
        if first:
            deg = agg[:, 16:17]
            deg_ref[...] = deg
        else:
            deg = deg_in_ref[...]
        agg = agg[:, :fout] / jnp.clip(deg, 1.0, None)
        r = agg + jnp.dot(h_ref[...], wroot_ref[...],
                          preferred_element_type=jnp.float32) + bias_ref[...]
        m = jnp.mean(r, axis=0, keepdims=True)
        v = jnp.mean((r - m) * (r - m), axis=0, keepdims=True)
        hn = gamma_ref[...] * (r - m) * lax.rsqrt(v + 1e-5) + beta_ref[...]
        hn = jnp.maximum(hn, 0.0)
        if out_w > fout:
            hn = jnp.concatenate(
                [hn, jnp.zeros((N, out_w - fout), jnp.float32)], axis=1)
        out_ref[...] = hn

    def run(scat, deg, h, wroot, bias, gamma, beta):
        full = lambda s: pl.BlockSpec(s, lambda: tuple(0 for _ in s))
        in_specs = [full((n_ch * 2 * ACC_R, 128))]
        args = [scat]
        if not first:
            in_specs.append(full((N, 1)))
            args.append(deg)
        in_specs += [full((N, 128)), full((128, fout)),
                     full((1, fout)), full((1, fout)), full((1, fout))]
        args += [h, wroot, bias, gamma, beta]
        out_shape = [jax.ShapeDtypeStruct((N, out_w), jnp.float32)]
        out_specs = [full((N, out_w))]
        if first:
            out_shape.append(jax.ShapeDtypeStruct((N, 1), jnp.float32))
            out_specs.append(full((N, 1)))
        res = pl.pallas_call(
            body,
            in_specs=in_specs,
            out_specs=out_specs,
            out_shape=out_shape,
        )(*args)
        return (res[0], res[1]) if first else (res[0], deg)

    return run


def _head_body(h_ref, wout_ref, bout_ref, out_ref):
    pooled = jnp.mean(h_ref[...], axis=0, keepdims=True)
    logits = jnp.dot(pooled, wout_ref[...],
                     preferred_element_type=jnp.float32) + bout_ref[...]
    mx = jnp.max(logits, axis=1, keepdims=True)
    lse = jnp.log(jnp.sum(jnp.exp(logits - mx), axis=1, keepdims=True)) + mx
    out_ref[...] = (logits - lse)[:, :10]


def _head(h, wout_p, bout_p, fin):
    full = lambda s: pl.BlockSpec(s, lambda: tuple(0 for _ in s))
    return pl.pallas_call(
        _head_body,
        in_specs=[full((N, fin)), full((fin, 128)), full((1, 128))],
        out_specs=full((1, 10)),
        out_shape=jax.ShapeDtypeStruct((1, 10), jnp.float32),
    )(h, wout_p, bout_p)


def kernel(x, edge_index, pseudo, params, Wout, bout):
    nl = len(params)
    src = edge_index[0]
    dst = edge_index[1]

    pad = EP - E
    src_p = jnp.concatenate([src, jnp.zeros((pad,), jnp.int32)])
    trash = N + (jnp.arange(pad, dtype=jnp.int32) % 16)
    dst_p = jnp.concatenate([dst, trash])
    src2d = src_p.reshape(EP // 128, 128)
    dst2d = dst_p.reshape(EP // 128, 128)
    pseudo_p = jnp.concatenate([pseudo, jnp.zeros((pad, 2), jnp.float32)])

    mu = jnp.stack([p["mu"] for p in params])
    sg = jnp.stack([p["sigma"] for p in params])
    mu0 = mu[:, :, 0].reshape(1, nl * K)
    mu1 = mu[:, :, 1].reshape(1, nl * K)
    s0 = sg[:, :, 0].reshape(1, nl * K)
    s1 = sg[:, :, 1].reshape(1, nl * K)
    gauss = _gauss_all(pseudo_p, mu0, mu1, s0, s1, nl)

    zeros = jnp.zeros((ACC_R, 128), jnp.float32)

    h = jnp.pad(x, ((0, 0), (0, 128 - x.shape[1])))
    deg = None
    for l, p in enumerate(params):
        fin = p["Wroot"].shape[0]
        fout = p["Wroot"].shape[1]
        n_ch = 2 if fout > 128 else 1
        wg = p["Wg"].reshape(fin, K, fout).transpose(1, 0, 2)
        wg = jnp.pad(wg, ((0, 0), (0, 128 - fin), (0, n_ch * 128 - fout)))
        wroot = jnp.pad(p["Wroot"], ((0, 128 - fin), (0, 0)))

        xj = _sc_gather(h, src2d)
        msg = _make_msg(l, n_ch, nl)(xj, gauss, wg)
        scat = (_sc_scatter_1 if n_ch == 1 else _sc_scatter_2)(msg, dst2d, zeros)
        h, deg = _make_finalize(l, n_ch, fout)(
            scat, deg, h, wroot,
            p["bias"].reshape(1, fout),
            p["gamma"].reshape(1, fout),
            p["beta"].reshape(1, fout),
        )

    wout_p = jnp.pad(Wout, ((0, 0), (0, 128 - Wout.shape[1])))
    bout_p = jnp.full((1, 128), -1e30, jnp.float32)
    bout_p = bout_p.at[0, :10].set(bout)
    return _head(h, wout_p, bout_p, h.shape[1])
